# baseline (speedup 1.0000x reference)
import jax
import jax.numpy as jnp
from jax import lax
from jax.experimental import pallas as pl
from jax.experimental.pallas import tpu as pltpu

B, C, S, D_MODEL = 4, 8, 128, 768
D_FF = 3072
E = 8
CS = C * S
N = B * CS
BF = 256
J = D_FF // BF


def _router_body(x_ref, bl_ref, spa_rW_ref, spa_rb_ref, spe_rW_ref, spe_rb_ref,
                 spa_b1_ref, spe_b1_ref, spa_b2_ref, spe_b2_ref, sh_b2_ref,
                 idx_a_ref, idx_b_ref, gate_a_ref, gate_b_ref,
                 b1a_ref, b1b_ref, b2tot_ref):
    inv = jnp.float32(1.0 / CS)
    xm = jnp.sum(x_ref[...].reshape(B, CS, D_MODEL), axis=1) * inv
    bm = jnp.sum(bl_ref[...].reshape(B, CS, D_MODEL), axis=1) * inv
    feats = jnp.concatenate([bm, xm, xm - bm], axis=-1)

    def route(rW, rb):
        logits = lax.dot_general(feats, rW, (((1,), (1,)), ((), ())),
                                 preferred_element_type=jnp.float32) + rb[0]
        p = jax.nn.softmax(logits, axis=-1)
        gate = jnp.max(p, axis=-1)
        idx = jnp.argmax(p, axis=-1).astype(jnp.int32)
        onehot = (jax.lax.broadcasted_iota(jnp.int32, (B, E), 1)
                  == idx[:, None]).astype(jnp.float32)
        return idx, gate, onehot

    idx_a, gate_a, oh_a = route(spa_rW_ref[...], spa_rb_ref[...])
    idx_b, gate_b, oh_b = route(spe_rW_ref[...], spe_rb_ref[...])

    idx_a_ref[...] = idx_a
    idx_b_ref[...] = idx_b
    gate_a_ref[...] = gate_a
    gate_b_ref[...] = gate_b
    b1a_ref[...] = (oh_a @ spa_b1_ref[...])[:, None, :]
    b1b_ref[...] = (oh_b @ spe_b1_ref[...])[:, None, :]
    b2tot = (sh_b2_ref[...]
             + gate_a[:, None] * (oh_a @ spa_b2_ref[...])
             + gate_b[:, None] * (oh_b @ spe_b2_ref[...]))
    b2tot_ref[...] = b2tot[:, None, :]


def _ffn_body(idx_a_ref, idx_b_ref, gate_a_ref, gate_b_ref,
              x_ref, w1s_ref, b1s_ref, w2s_ref,
              w1a0_ref, w1a1_ref, w1a2_ref, w1a3_ref,
              w2a0_ref, w2a1_ref, w2a2_ref, w2a3_ref,
              w1b0_ref, w1b1_ref, w1b2_ref, w1b3_ref,
              w2b0_ref, w2b1_ref, w2b2_ref, w2b3_ref,
              b1a_ref, b1b_ref, b2tot_ref, o_ref):
    j = pl.program_id(0)
    cdims = (((1,), (1,)), ((), ()))

    def mm(a, w):
        return lax.dot_general(a, w, cdims, preferred_element_type=jnp.float32)

    @pl.when(j == 0)
    def _init():
        o_ref[...] = jnp.broadcast_to(
            b2tot_ref[...].reshape(B, 1, D_MODEL), (B, CS, D_MODEL)
        ).reshape(N, D_MODEL)

    x = x_ref[...]
    h_s = jax.nn.gelu(mm(x, w1s_ref[...]) + b1s_ref[0, 0, :])
    o_ref[...] += mm(h_s, w2s_ref[...])

    w1a = (w1a0_ref, w1a1_ref, w1a2_ref, w1a3_ref)
    w2a = (w2a0_ref, w2a1_ref, w2a2_ref, w2a3_ref)
    w1b = (w1b0_ref, w1b1_ref, w1b2_ref, w1b3_ref)
    w2b = (w2b0_ref, w2b1_ref, w2b2_ref, w2b3_ref)
    for b in range(B):
        xb = x_ref[b * CS:(b + 1) * CS, :]
        h_a = jax.nn.gelu(mm(xb, w1a[b][0]) + b1a_ref[b, 0, :]) * gate_a_ref[b]
        h_b = jax.nn.gelu(mm(xb, w1b[b][0]) + b1b_ref[b, 0, :]) * gate_b_ref[b]
        o_ref[b * CS:(b + 1) * CS, :] += mm(h_a, w2a[b][0]) + mm(h_b, w2b[b][0])


@jax.jit
def kernel(x, baseline, shared_W1, shared_b1, shared_W2, shared_b2,
           spa_rW, spa_rb, spa_W1, spa_b1, spa_W2, spa_b2,
           spe_rW, spe_rb, spe_W1, spe_b1, spe_W2, spe_b2):
    f32 = jnp.float32
    x3 = x.reshape(B, CS, D_MODEL)
    bl3 = baseline.reshape(B, CS, D_MODEL)

    router_out = pl.pallas_call(
        _router_body,
        out_shape=(
            jax.ShapeDtypeStruct((B,), jnp.int32),
            jax.ShapeDtypeStruct((B,), jnp.int32),
            jax.ShapeDtypeStruct((B,), f32),
            jax.ShapeDtypeStruct((B,), f32),
            jax.ShapeDtypeStruct((B, 1, D_FF), f32),
            jax.ShapeDtypeStruct((B, 1, D_FF), f32),
            jax.ShapeDtypeStruct((B, 1, D_MODEL), f32),
        ),
    )(x3, bl3, spa_rW, spa_rb.reshape(1, E), spe_rW, spe_rb.reshape(1, E),
      spa_b1, spe_b1, spa_b2, spe_b2, shared_b2.reshape(1, D_MODEL))

    idx_a, idx_b, gate_a, gate_b, b1a, b1b, b2tot = router_out

    def w1_spec(idx_no, b):
        def im(j, ia, ib, ga, gb):
            sel = (ia, ib)[idx_no]
            return (sel[b], j, 0)
        return pl.BlockSpec((1, BF, D_MODEL), im)

    def w2_spec(idx_no, b):
        def im(j, ia, ib, ga, gb):
            sel = (ia, ib)[idx_no]
            return (sel[b], 0, j)
        return pl.BlockSpec((1, D_MODEL, BF), im)

    grid_spec = pltpu.PrefetchScalarGridSpec(
        num_scalar_prefetch=4,
        grid=(J,),
        in_specs=[
            pl.BlockSpec((N, D_MODEL), lambda j, ia, ib, ga, gb: (0, 0)),
            pl.BlockSpec((BF, D_MODEL), lambda j, ia, ib, ga, gb: (j, 0)),
            pl.BlockSpec((1, 1, BF), lambda j, ia, ib, ga, gb: (0, 0, j)),
            pl.BlockSpec((D_MODEL, BF), lambda j, ia, ib, ga, gb: (0, j)),
            *[w1_spec(0, b) for b in range(B)],
            *[w2_spec(0, b) for b in range(B)],
            *[w1_spec(1, b) for b in range(B)],
            *[w2_spec(1, b) for b in range(B)],
            pl.BlockSpec((B, 1, BF), lambda j, ia, ib, ga, gb: (0, 0, j)),
            pl.BlockSpec((B, 1, BF), lambda j, ia, ib, ga, gb: (0, 0, j)),
            pl.BlockSpec((B, 1, D_MODEL), lambda j, ia, ib, ga, gb: (0, 0, 0)),
        ],
        out_specs=pl.BlockSpec((N, D_MODEL), lambda j, ia, ib, ga, gb: (0, 0)),
    )

    out = pl.pallas_call(
        _ffn_body,
        grid_spec=grid_spec,
        out_shape=jax.ShapeDtypeStruct((N, D_MODEL), f32),
        compiler_params=pltpu.CompilerParams(
            dimension_semantics=("arbitrary",)),
    )(idx_a, idx_b, gate_a, gate_b,
      x3.reshape(N, D_MODEL), shared_W1, shared_b1.reshape(1, 1, D_FF),
      shared_W2,
      spa_W1, spa_W1, spa_W1, spa_W1, spa_W2, spa_W2, spa_W2, spa_W2,
      spe_W1, spe_W1, spe_W1, spe_W1, spe_W2, spe_W2, spe_W2, spe_W2,
      b1a, b1b, b2tot)

    return out.reshape(B, C, S, D_MODEL)


# speedup vs baseline: 1.1076x; 1.1076x over previous
import jax
import jax.numpy as jnp
from jax import lax
from jax.experimental import pallas as pl
from jax.experimental.pallas import tpu as pltpu

B, C, S, D_MODEL = 4, 8, 128, 768
D_FF = 3072
E = 8
CS = C * S
BF = 512
J = D_FF // BF


def _router_body(x_ref, bl_ref, spa_rW_ref, spa_rb_ref, spe_rW_ref, spe_rb_ref,
                 spa_b1_ref, spe_b1_ref, spa_b2_ref, spe_b2_ref, sh_b2_ref,
                 idx_a_ref, idx_b_ref, gate_a_ref, gate_b_ref,
                 b1a_ref, b1b_ref, b2tot_ref):
    inv = jnp.float32(1.0 / CS)
    xm = jnp.sum(x_ref[...].reshape(B, CS, D_MODEL), axis=1) * inv
    bm = jnp.sum(bl_ref[...].reshape(B, CS, D_MODEL), axis=1) * inv
    feats = jnp.concatenate([bm, xm, xm - bm], axis=-1)

    def route(rW, rb):
        logits = lax.dot_general(feats, rW, (((1,), (1,)), ((), ())),
                                 preferred_element_type=jnp.float32) + rb[0]
        p = jax.nn.softmax(logits, axis=-1)
        gate = jnp.max(p, axis=-1)
        idx = jnp.argmax(p, axis=-1).astype(jnp.int32)
        onehot = (jax.lax.broadcasted_iota(jnp.int32, (B, E), 1)
                  == idx[:, None]).astype(jnp.float32)
        return idx, gate, onehot

    idx_a, gate_a, oh_a = route(spa_rW_ref[...], spa_rb_ref[...])
    idx_b, gate_b, oh_b = route(spe_rW_ref[...], spe_rb_ref[...])

    idx_a_ref[...] = idx_a
    idx_b_ref[...] = idx_b
    gate_a_ref[...] = gate_a
    gate_b_ref[...] = gate_b
    b1a_ref[...] = (oh_a @ spa_b1_ref[...])[:, None, :]
    b1b_ref[...] = (oh_b @ spe_b1_ref[...])[:, None, :]
    b2tot = (sh_b2_ref[...]
             + gate_a[:, None] * (oh_a @ spa_b2_ref[...])
             + gate_b[:, None] * (oh_b @ spe_b2_ref[...]))
    b2tot_ref[...] = b2tot[:, None, :]


def _ffn_body(idx_a_ref, idx_b_ref, gate_a_ref, gate_b_ref,
              x_ref, w1s_ref, b1s_ref, w2s_ref,
              w1a_ref, w2a_ref, w1b_ref, w2b_ref,
              b1a_ref, b1b_ref, b2tot_ref, o_ref):
    b = pl.program_id(0)
    j = pl.program_id(1)
    bf16 = jnp.bfloat16
    x = x_ref[0].astype(bf16)
    ga = gate_a_ref[b]
    gb = gate_b_ref[b]
    cdims = (((1,), (1,)), ((), ()))

    def mm(a, w):
        return lax.dot_general(a, w.astype(bf16), cdims,
                               preferred_element_type=jnp.float32)

    h_s = jax.nn.gelu(mm(x, w1s_ref[...]) + b1s_ref[0, 0, :]).astype(bf16)
    h_a = (jax.nn.gelu(mm(x, w1a_ref[0]) + b1a_ref[0, 0, :]) * ga).astype(bf16)
    h_b = (jax.nn.gelu(mm(x, w1b_ref[0]) + b1b_ref[0, 0, :]) * gb).astype(bf16)

    acc = mm(h_s, w2s_ref[...]) + mm(h_a, w2a_ref[0]) + mm(h_b, w2b_ref[0])

    @pl.when(j == 0)
    def _init():
        o_ref[0] = acc + b2tot_ref[0, 0, :]

    @pl.when(j > 0)
    def _acc():
        o_ref[0] += acc


@jax.jit
def kernel(x, baseline, shared_W1, shared_b1, shared_W2, shared_b2,
           spa_rW, spa_rb, spa_W1, spa_b1, spa_W2, spa_b2,
           spe_rW, spe_rb, spe_W1, spe_b1, spe_W2, spe_b2):
    f32 = jnp.float32
    x3 = x.reshape(B, CS, D_MODEL)
    bl3 = baseline.reshape(B, CS, D_MODEL)

    router_out = pl.pallas_call(
        _router_body,
        out_shape=(
            jax.ShapeDtypeStruct((B,), jnp.int32),
            jax.ShapeDtypeStruct((B,), jnp.int32),
            jax.ShapeDtypeStruct((B,), f32),
            jax.ShapeDtypeStruct((B,), f32),
            jax.ShapeDtypeStruct((B, 1, D_FF), f32),
            jax.ShapeDtypeStruct((B, 1, D_FF), f32),
            jax.ShapeDtypeStruct((B, 1, D_MODEL), f32),
        ),
    )(x3, bl3, spa_rW, spa_rb.reshape(1, E), spe_rW, spe_rb.reshape(1, E),
      spa_b1, spe_b1, spa_b2, spe_b2, shared_b2.reshape(1, D_MODEL))

    idx_a, idx_b, gate_a, gate_b, b1a, b1b, b2tot = router_out

    grid_spec = pltpu.PrefetchScalarGridSpec(
        num_scalar_prefetch=4,
        grid=(B, J),
        in_specs=[
            pl.BlockSpec((1, CS, D_MODEL), lambda b, j, ia, ib, ga, gb: (b, 0, 0)),
            pl.BlockSpec((BF, D_MODEL), lambda b, j, ia, ib, ga, gb: (j, 0)),
            pl.BlockSpec((1, 1, BF), lambda b, j, ia, ib, ga, gb: (0, 0, j)),
            pl.BlockSpec((D_MODEL, BF), lambda b, j, ia, ib, ga, gb: (0, j)),
            pl.BlockSpec((1, BF, D_MODEL),
                         lambda b, j, ia, ib, ga, gb: (ia[b], j, 0)),
            pl.BlockSpec((1, D_MODEL, BF),
                         lambda b, j, ia, ib, ga, gb: (ia[b], 0, j)),
            pl.BlockSpec((1, BF, D_MODEL),
                         lambda b, j, ia, ib, ga, gb: (ib[b], j, 0)),
            pl.BlockSpec((1, D_MODEL, BF),
                         lambda b, j, ia, ib, ga, gb: (ib[b], 0, j)),
            pl.BlockSpec((1, 1, BF), lambda b, j, ia, ib, ga, gb: (b, 0, j)),
            pl.BlockSpec((1, 1, BF), lambda b, j, ia, ib, ga, gb: (b, 0, j)),
            pl.BlockSpec((1, 1, D_MODEL), lambda b, j, ia, ib, ga, gb: (b, 0, 0)),
        ],
        out_specs=pl.BlockSpec((1, CS, D_MODEL),
                               lambda b, j, ia, ib, ga, gb: (b, 0, 0)),
    )

    out = pl.pallas_call(
        _ffn_body,
        grid_spec=grid_spec,
        out_shape=jax.ShapeDtypeStruct((B, CS, D_MODEL), f32),
        compiler_params=pltpu.CompilerParams(
            dimension_semantics=("arbitrary", "arbitrary")),
    )(idx_a, idx_b, gate_a, gate_b,
      x3, shared_W1, shared_b1.reshape(1, 1, D_FF), shared_W2,
      spa_W1, spa_W2, spe_W1, spe_W2, b1a, b1b, b2tot)

    return out.reshape(B, C, S, D_MODEL)


# speedup vs baseline: 1.1973x; 1.0810x over previous
import jax
import jax.numpy as jnp
from jax import lax
from jax.experimental import pallas as pl
from jax.experimental.pallas import tpu as pltpu

B, C, S, D_MODEL = 4, 8, 128, 768
D_FF = 3072
E = 8
CS = C * S
BF = 768
J = D_FF // BF


def _router_body(x_ref, bl_ref, spa_rW_ref, spa_rb_ref, spe_rW_ref, spe_rb_ref,
                 spa_b1_ref, spe_b1_ref, spa_b2_ref, spe_b2_ref, sh_b2_ref,
                 idx_a_ref, idx_b_ref, gate_a_ref, gate_b_ref,
                 b1a_ref, b1b_ref, b2tot_ref):
    inv = jnp.float32(1.0 / CS)
    xm = jnp.sum(x_ref[...].reshape(B, CS, D_MODEL), axis=1) * inv
    bm = jnp.sum(bl_ref[...].reshape(B, CS, D_MODEL), axis=1) * inv
    feats = jnp.concatenate([bm, xm, xm - bm], axis=-1)

    def route(rW, rb):
        logits = lax.dot_general(feats, rW, (((1,), (1,)), ((), ())),
                                 preferred_element_type=jnp.float32) + rb[0]
        p = jax.nn.softmax(logits, axis=-1)
        gate = jnp.max(p, axis=-1)
        idx = jnp.argmax(p, axis=-1).astype(jnp.int32)
        onehot = (jax.lax.broadcasted_iota(jnp.int32, (B, E), 1)
                  == idx[:, None]).astype(jnp.float32)
        return idx, gate, onehot

    idx_a, gate_a, oh_a = route(spa_rW_ref[...], spa_rb_ref[...])
    idx_b, gate_b, oh_b = route(spe_rW_ref[...], spe_rb_ref[...])

    idx_a_ref[...] = idx_a
    idx_b_ref[...] = idx_b
    gate_a_ref[...] = gate_a
    gate_b_ref[...] = gate_b
    b1a_ref[...] = (oh_a @ spa_b1_ref[...])[:, None, :]
    b1b_ref[...] = (oh_b @ spe_b1_ref[...])[:, None, :]
    b2tot = (sh_b2_ref[...]
             + gate_a[:, None] * (oh_a @ spa_b2_ref[...])
             + gate_b[:, None] * (oh_b @ spe_b2_ref[...]))
    b2tot_ref[...] = b2tot[:, None, :]


def _ffn_body(idx_a_ref, idx_b_ref, gate_a_ref, gate_b_ref,
              x_ref, w1s_ref, b1s_ref, w2s_ref,
              w1a_ref, w2a_ref, w1b_ref, w2b_ref,
              b1a_ref, b1b_ref, b2tot_ref, o_ref):
    b = pl.program_id(0)
    j = pl.program_id(1)
    x = x_ref[0]
    ga = gate_a_ref[b]
    gb = gate_b_ref[b]
    cdims = (((1,), (1,)), ((), ()))

    def mm(a, w):
        return lax.dot_general(a, w, cdims, preferred_element_type=jnp.float32)

    h_s = jax.nn.gelu(mm(x, w1s_ref[...]) + b1s_ref[0, 0, :])
    h_a = jax.nn.gelu(mm(x, w1a_ref[0]) + b1a_ref[0, 0, :]) * ga
    h_b = jax.nn.gelu(mm(x, w1b_ref[0]) + b1b_ref[0, 0, :]) * gb

    acc = mm(h_s, w2s_ref[...]) + mm(h_a, w2a_ref[0]) + mm(h_b, w2b_ref[0])

    @pl.when(j == 0)
    def _init():
        o_ref[0] = acc + b2tot_ref[0, 0, :]

    @pl.when(j > 0)
    def _acc():
        o_ref[0] += acc


@jax.jit
def kernel(x, baseline, shared_W1, shared_b1, shared_W2, shared_b2,
           spa_rW, spa_rb, spa_W1, spa_b1, spa_W2, spa_b2,
           spe_rW, spe_rb, spe_W1, spe_b1, spe_W2, spe_b2):
    f32 = jnp.float32
    x3 = x.reshape(B, CS, D_MODEL)
    bl3 = baseline.reshape(B, CS, D_MODEL)

    router_out = pl.pallas_call(
        _router_body,
        out_shape=(
            jax.ShapeDtypeStruct((B,), jnp.int32),
            jax.ShapeDtypeStruct((B,), jnp.int32),
            jax.ShapeDtypeStruct((B,), f32),
            jax.ShapeDtypeStruct((B,), f32),
            jax.ShapeDtypeStruct((B, 1, D_FF), f32),
            jax.ShapeDtypeStruct((B, 1, D_FF), f32),
            jax.ShapeDtypeStruct((B, 1, D_MODEL), f32),
        ),
    )(x3, bl3, spa_rW, spa_rb.reshape(1, E), spe_rW, spe_rb.reshape(1, E),
      spa_b1, spe_b1, spa_b2, spe_b2, shared_b2.reshape(1, D_MODEL))

    idx_a, idx_b, gate_a, gate_b, b1a, b1b, b2tot = router_out

    grid_spec = pltpu.PrefetchScalarGridSpec(
        num_scalar_prefetch=4,
        grid=(B, J),
        in_specs=[
            pl.BlockSpec((1, CS, D_MODEL), lambda b, j, ia, ib, ga, gb: (b, 0, 0)),
            pl.BlockSpec((BF, D_MODEL), lambda b, j, ia, ib, ga, gb: (j, 0)),
            pl.BlockSpec((1, 1, BF), lambda b, j, ia, ib, ga, gb: (0, 0, j)),
            pl.BlockSpec((D_MODEL, BF), lambda b, j, ia, ib, ga, gb: (0, j)),
            pl.BlockSpec((1, BF, D_MODEL),
                         lambda b, j, ia, ib, ga, gb: (ia[b], j, 0)),
            pl.BlockSpec((1, D_MODEL, BF),
                         lambda b, j, ia, ib, ga, gb: (ia[b], 0, j)),
            pl.BlockSpec((1, BF, D_MODEL),
                         lambda b, j, ia, ib, ga, gb: (ib[b], j, 0)),
            pl.BlockSpec((1, D_MODEL, BF),
                         lambda b, j, ia, ib, ga, gb: (ib[b], 0, j)),
            pl.BlockSpec((1, 1, BF), lambda b, j, ia, ib, ga, gb: (b, 0, j)),
            pl.BlockSpec((1, 1, BF), lambda b, j, ia, ib, ga, gb: (b, 0, j)),
            pl.BlockSpec((1, 1, D_MODEL), lambda b, j, ia, ib, ga, gb: (b, 0, 0)),
        ],
        out_specs=pl.BlockSpec((1, CS, D_MODEL),
                               lambda b, j, ia, ib, ga, gb: (b, 0, 0)),
    )

    out = pl.pallas_call(
        _ffn_body,
        grid_spec=grid_spec,
        out_shape=jax.ShapeDtypeStruct((B, CS, D_MODEL), f32),
        compiler_params=pltpu.CompilerParams(
            dimension_semantics=("arbitrary", "arbitrary")),
    )(idx_a, idx_b, gate_a, gate_b,
      x3, shared_W1, shared_b1.reshape(1, 1, D_FF), shared_W2,
      spa_W1, spa_W2, spe_W1, spe_W2, b1a, b1b, b2tot)

    return out.reshape(B, C, S, D_MODEL)


# speedup vs baseline: 1.2010x; 1.0031x over previous
import jax
import jax.numpy as jnp
from jax import lax
from jax.experimental import pallas as pl
from jax.experimental.pallas import tpu as pltpu

B, C, S, D_MODEL = 4, 8, 128, 768
D_FF = 3072
E = 8
CS = C * S
BF = 1024
J = D_FF // BF


def _router_body(x_ref, bl_ref, spa_rW_ref, spa_rb_ref, spe_rW_ref, spe_rb_ref,
                 spa_b1_ref, spe_b1_ref, spa_b2_ref, spe_b2_ref, sh_b2_ref,
                 idx_a_ref, idx_b_ref, gate_a_ref, gate_b_ref,
                 b1a_ref, b1b_ref, b2tot_ref):
    inv = jnp.float32(1.0 / CS)
    xm = jnp.sum(x_ref[...].reshape(B, CS, D_MODEL), axis=1) * inv
    bm = jnp.sum(bl_ref[...].reshape(B, CS, D_MODEL), axis=1) * inv
    feats = jnp.concatenate([bm, xm, xm - bm], axis=-1)

    def route(rW, rb):
        logits = lax.dot_general(feats, rW, (((1,), (1,)), ((), ())),
                                 preferred_element_type=jnp.float32) + rb[0]
        p = jax.nn.softmax(logits, axis=-1)
        gate = jnp.max(p, axis=-1)
        idx = jnp.argmax(p, axis=-1).astype(jnp.int32)
        onehot = (jax.lax.broadcasted_iota(jnp.int32, (B, E), 1)
                  == idx[:, None]).astype(jnp.float32)
        return idx, gate, onehot

    idx_a, gate_a, oh_a = route(spa_rW_ref[...], spa_rb_ref[...])
    idx_b, gate_b, oh_b = route(spe_rW_ref[...], spe_rb_ref[...])

    idx_a_ref[...] = idx_a
    idx_b_ref[...] = idx_b
    gate_a_ref[...] = gate_a
    gate_b_ref[...] = gate_b
    b1a_ref[...] = (oh_a @ spa_b1_ref[...])[:, None, :]
    b1b_ref[...] = (oh_b @ spe_b1_ref[...])[:, None, :]
    b2tot = (sh_b2_ref[...]
             + gate_a[:, None] * (oh_a @ spa_b2_ref[...])
             + gate_b[:, None] * (oh_b @ spe_b2_ref[...]))
    b2tot_ref[...] = b2tot[:, None, :]


def _ffn_body(idx_a_ref, idx_b_ref, gate_a_ref, gate_b_ref,
              x_ref, w1s_ref, b1s_ref, w2s_ref,
              w1a_ref, w2a_ref, w1b_ref, w2b_ref,
              b1a_ref, b1b_ref, b2tot_ref, o_ref):
    b = pl.program_id(0)
    j = pl.program_id(1)
    x = x_ref[0]
    ga = gate_a_ref[b]
    gb = gate_b_ref[b]
    cdims = (((1,), (1,)), ((), ()))

    def mm(a, w):
        return lax.dot_general(a, w, cdims, preferred_element_type=jnp.float32)

    h_s = jax.nn.gelu(mm(x, w1s_ref[...]) + b1s_ref[0, 0, :])
    h_a = jax.nn.gelu(mm(x, w1a_ref[0]) + b1a_ref[0, 0, :]) * ga
    h_b = jax.nn.gelu(mm(x, w1b_ref[0]) + b1b_ref[0, 0, :]) * gb

    acc = mm(h_s, w2s_ref[...]) + mm(h_a, w2a_ref[0]) + mm(h_b, w2b_ref[0])

    @pl.when(j == 0)
    def _init():
        o_ref[0] = acc + b2tot_ref[0, 0, :]

    @pl.when(j > 0)
    def _acc():
        o_ref[0] += acc


@jax.jit
def kernel(x, baseline, shared_W1, shared_b1, shared_W2, shared_b2,
           spa_rW, spa_rb, spa_W1, spa_b1, spa_W2, spa_b2,
           spe_rW, spe_rb, spe_W1, spe_b1, spe_W2, spe_b2):
    f32 = jnp.float32
    x3 = x.reshape(B, CS, D_MODEL)
    bl3 = baseline.reshape(B, CS, D_MODEL)

    router_out = pl.pallas_call(
        _router_body,
        out_shape=(
            jax.ShapeDtypeStruct((B,), jnp.int32),
            jax.ShapeDtypeStruct((B,), jnp.int32),
            jax.ShapeDtypeStruct((B,), f32),
            jax.ShapeDtypeStruct((B,), f32),
            jax.ShapeDtypeStruct((B, 1, D_FF), f32),
            jax.ShapeDtypeStruct((B, 1, D_FF), f32),
            jax.ShapeDtypeStruct((B, 1, D_MODEL), f32),
        ),
    )(x3, bl3, spa_rW, spa_rb.reshape(1, E), spe_rW, spe_rb.reshape(1, E),
      spa_b1, spe_b1, spa_b2, spe_b2, shared_b2.reshape(1, D_MODEL))

    idx_a, idx_b, gate_a, gate_b, b1a, b1b, b2tot = router_out

    grid_spec = pltpu.PrefetchScalarGridSpec(
        num_scalar_prefetch=4,
        grid=(B, J),
        in_specs=[
            pl.BlockSpec((1, CS, D_MODEL), lambda b, j, ia, ib, ga, gb: (b, 0, 0)),
            pl.BlockSpec((BF, D_MODEL), lambda b, j, ia, ib, ga, gb: (j, 0)),
            pl.BlockSpec((1, 1, BF), lambda b, j, ia, ib, ga, gb: (0, 0, j)),
            pl.BlockSpec((D_MODEL, BF), lambda b, j, ia, ib, ga, gb: (0, j)),
            pl.BlockSpec((1, BF, D_MODEL),
                         lambda b, j, ia, ib, ga, gb: (ia[b], j, 0)),
            pl.BlockSpec((1, D_MODEL, BF),
                         lambda b, j, ia, ib, ga, gb: (ia[b], 0, j)),
            pl.BlockSpec((1, BF, D_MODEL),
                         lambda b, j, ia, ib, ga, gb: (ib[b], j, 0)),
            pl.BlockSpec((1, D_MODEL, BF),
                         lambda b, j, ia, ib, ga, gb: (ib[b], 0, j)),
            pl.BlockSpec((1, 1, BF), lambda b, j, ia, ib, ga, gb: (b, 0, j)),
            pl.BlockSpec((1, 1, BF), lambda b, j, ia, ib, ga, gb: (b, 0, j)),
            pl.BlockSpec((1, 1, D_MODEL), lambda b, j, ia, ib, ga, gb: (b, 0, 0)),
        ],
        out_specs=pl.BlockSpec((1, CS, D_MODEL),
                               lambda b, j, ia, ib, ga, gb: (b, 0, 0)),
    )

    out = pl.pallas_call(
        _ffn_body,
        grid_spec=grid_spec,
        out_shape=jax.ShapeDtypeStruct((B, CS, D_MODEL), f32),
        compiler_params=pltpu.CompilerParams(
            dimension_semantics=("arbitrary", "arbitrary"),
            vmem_limit_bytes=100 * 1024 * 1024),
    )(idx_a, idx_b, gate_a, gate_b,
      x3, shared_W1, shared_b1.reshape(1, 1, D_FF), shared_W2,
      spa_W1, spa_W2, spe_W1, spe_W2, b1a, b1b, b2tot)

    return out.reshape(B, C, S, D_MODEL)
